# baseline (speedup 1.0000x reference)
import jax
import jax.numpy as jnp
from jax import lax
from jax.experimental import pallas as pl
from jax.experimental.pallas import tpu as pltpu
from jax.experimental.pallas import tpu_sc as plsc

NC, NS = 2, 16
NW = NC * NS
RDUP = 8
CHS = 256
NBUF = 3
DEPTH = 2


def _sc_body(kc_ref, vc_ref, krows_ref, vrows_ref, idx_ref, ok_ref, ov_ref,
             buf, krow_v, vrow_v, idx_v, sems_in, sems_out, sem_row):
    wid = lax.axis_index("s") * NC + lax.axis_index("c")
    rpw = 4
    base = wid * rpw

    pltpu.sync_copy(krows_ref.at[wid], krow_v)
    pltpu.sync_copy(vrows_ref.at[wid], vrow_v)
    pltpu.sync_copy(idx_ref.at[wid], idx_v)

    S = 2048
    nch = S // CHS
    n_chunks = 2 * rpw * nch

    def src_dst(i):
        cache = i // (rpw * nch)
        r = (i % (rpw * nch)) // nch
        ch = i % nch
        row0 = (base + r) * S + ch * CHS
        src = kc_ref if cache == 0 else vc_ref
        dst = ok_ref if cache == 0 else ov_ref
        return src.at[pl.ds(row0, CHS)], dst.at[pl.ds(row0, CHS)]

    in_cp, out_cp = {}, {}

    def drain(j):
        slot = j % NBUF
        in_cp[j].wait()
        _, dst = src_dst(j)
        cp = pltpu.make_async_copy(buf.at[slot], dst, sems_out.at[slot])
        cp.start()
        out_cp[j] = cp

    for i in range(n_chunks):
        slot = i % NBUF
        if i >= NBUF:
            out_cp[i - NBUF].wait()
        src, _ = src_dst(i)
        cp = pltpu.make_async_copy(src, buf.at[slot], sems_in.at[slot])
        cp.start()
        in_cp[i] = cp
        if i >= DEPTH:
            drain(i - DEPTH)
    for j in range(n_chunks - DEPTH, n_chunks):
        drain(j)
    for j in range(n_chunks - NBUF, n_chunks):
        out_cp[j].wait()

    sc_k = pltpu.make_async_copy(krow_v, ok_ref.at[idx_v], sem_row)
    sc_k.start()
    sc_k.wait()
    sc_v = pltpu.make_async_copy(vrow_v, ov_ref.at[idx_v], sem_row)
    sc_v.start()
    sc_v.wait()


def kernel(k, k_cache, v, v_cache, n_tokens):
    B, H, S, D = k_cache.shape
    BH = B * H
    nt = jnp.asarray(n_tokens, jnp.int32)
    kcf = k_cache.reshape(BH * S, D)
    vcf = v_cache.reshape(BH * S, D)
    rows_k = k.reshape(NW, BH // NW, D)
    rows_v = v.reshape(NW, BH // NW, D)
    krows = jnp.concatenate([rows_k, rows_k], axis=1)
    vrows = jnp.concatenate([rows_v, rows_v], axis=1)
    flat = jnp.arange(BH, dtype=jnp.int32).reshape(NW, BH // NW) * S + nt
    idx = jnp.concatenate([flat, flat], axis=1)

    mesh = plsc.VectorSubcoreMesh(core_axis_name="c", subcore_axis_name="s")
    run = pl.kernel(
        _sc_body,
        mesh=mesh,
        out_type=[
            jax.ShapeDtypeStruct((BH * S, D), k_cache.dtype),
            jax.ShapeDtypeStruct((BH * S, D), v_cache.dtype),
        ],
        scratch_types=[
            pltpu.VMEM((NBUF, CHS, D), k_cache.dtype),
            pltpu.VMEM((RDUP, D), k.dtype),
            pltpu.VMEM((RDUP, D), v.dtype),
            pltpu.VMEM((RDUP,), jnp.int32),
            pltpu.SemaphoreType.DMA((NBUF,)),
            pltpu.SemaphoreType.DMA((NBUF,)),
            pltpu.SemaphoreType.DMA,
        ],
    )
    out_k, out_v = run(kcf, vcf, krows, vrows, idx)
    return (out_k.reshape(B, H, S, D), out_v.reshape(B, H, S, D))


# speedup vs baseline: 1.0772x; 1.0772x over previous
import jax
import jax.numpy as jnp
from jax import lax
from jax.experimental import pallas as pl
from jax.experimental.pallas import tpu as pltpu
from jax.experimental.pallas import tpu_sc as plsc

CBH = 2
NBUF = 12
DEPTH = 6

NC, NS = 2, 16
NW = NC * NS
RDUP = 8
CHS = 256
SC_NBUF = 3
SC_DEPTH = 2


def _tc_body(nt_ref, k_ref, kc_ref, ok_ref, bufs, krows, sem_rows,
             sems_in, sems_out):
    BH, S, D = kc_ref.shape
    n_chunks = BH // CBH
    nt = nt_ref[0]

    ld = pltpu.make_async_copy(k_ref, krows, sem_rows)
    ld.start()
    ld.wait()

    def start_in(i):
        cp = pltpu.make_async_copy(kc_ref.at[pl.ds(i * CBH, CBH)],
                                   bufs.at[i % NBUF], sems_in.at[i % NBUF])
        cp.start()
        return cp

    def drain(j, in_copies):
        slot = j % NBUF
        in_copies[j].wait()
        for c in range(CBH):
            bufs[slot, c, pl.ds(nt, 1), :] = krows[j * CBH + c]
        cp = pltpu.make_async_copy(bufs.at[slot],
                                   ok_ref.at[pl.ds(j * CBH, CBH)],
                                   sems_out.at[slot])
        cp.start()
        return cp

    in_copies, out_copies = {}, {}
    for i in range(n_chunks):
        if i >= NBUF:
            out_copies[i - NBUF].wait()
        in_copies[i] = start_in(i)
        if i >= DEPTH:
            out_copies[i - DEPTH] = drain(i - DEPTH, in_copies)
    for j in range(n_chunks - DEPTH, n_chunks):
        out_copies[j] = drain(j, in_copies)
    for j in range(max(0, n_chunks - NBUF), n_chunks):
        out_copies[j].wait()


def _sc_body(vc_ref, vrows_ref, idx_ref, ov_ref,
             buf, vrow_v, idx_v, sems_in, sems_out, sem_row):
    wid = lax.axis_index("s") * NC + lax.axis_index("c")
    rpw = 4
    base = wid * rpw

    pltpu.sync_copy(vrows_ref.at[wid], vrow_v)
    pltpu.sync_copy(idx_ref.at[wid], idx_v)

    S = 2048
    nch = S // CHS
    n_chunks = rpw * nch

    def src_dst(i):
        row0 = (base + i // nch) * S + (i % nch) * CHS
        return vc_ref.at[pl.ds(row0, CHS)], ov_ref.at[pl.ds(row0, CHS)]

    in_cp, out_cp = {}, {}

    def drain(j):
        slot = j % SC_NBUF
        in_cp[j].wait()
        _, dst = src_dst(j)
        cp = pltpu.make_async_copy(buf.at[slot], dst, sems_out.at[slot])
        cp.start()
        out_cp[j] = cp

    for i in range(n_chunks):
        slot = i % SC_NBUF
        if i >= SC_NBUF:
            out_cp[i - SC_NBUF].wait()
        src, _ = src_dst(i)
        cp = pltpu.make_async_copy(src, buf.at[slot], sems_in.at[slot])
        cp.start()
        in_cp[i] = cp
        if i >= SC_DEPTH:
            drain(i - SC_DEPTH)
    for j in range(n_chunks - SC_DEPTH, n_chunks):
        drain(j)
    for j in range(n_chunks - SC_NBUF, n_chunks):
        out_cp[j].wait()

    sc_v = pltpu.make_async_copy(vrow_v, ov_ref.at[idx_v], sem_row)
    sc_v.start()
    sc_v.wait()


def kernel(k, k_cache, v, v_cache, n_tokens):
    B, H, S, D = k_cache.shape
    BH = B * H
    nt = jnp.asarray(n_tokens, jnp.int32)

    vcf = v_cache.reshape(BH * S, D)
    rows_v = v.reshape(NW, BH // NW, D)
    vrows = jnp.concatenate([rows_v, rows_v], axis=1)
    flat = jnp.arange(BH, dtype=jnp.int32).reshape(NW, BH // NW) * S + nt
    idx = jnp.concatenate([flat, flat], axis=1)

    mesh = plsc.VectorSubcoreMesh(core_axis_name="c", subcore_axis_name="s")
    sc_run = pl.kernel(
        _sc_body,
        mesh=mesh,
        out_type=jax.ShapeDtypeStruct((BH * S, D), v_cache.dtype),
        scratch_types=[
            pltpu.VMEM((SC_NBUF, CHS, D), v_cache.dtype),
            pltpu.VMEM((RDUP, D), v.dtype),
            pltpu.VMEM((RDUP,), jnp.int32),
            pltpu.SemaphoreType.DMA((SC_NBUF,)),
            pltpu.SemaphoreType.DMA((SC_NBUF,)),
            pltpu.SemaphoreType.DMA,
        ],
    )
    out_v = sc_run(vcf, vrows, idx)

    any_spec = pl.BlockSpec(memory_space=pl.ANY)
    out_k = pl.pallas_call(
        _tc_body,
        in_specs=[
            pl.BlockSpec(memory_space=pltpu.SMEM),
            any_spec, any_spec,
        ],
        out_specs=any_spec,
        out_shape=jax.ShapeDtypeStruct((BH, S, D), k_cache.dtype),
        scratch_shapes=[
            pltpu.VMEM((NBUF, CBH, S, D), k_cache.dtype),
            pltpu.VMEM((BH, 1, D), k.dtype),
            pltpu.SemaphoreType.DMA,
            pltpu.SemaphoreType.DMA((NBUF,)),
            pltpu.SemaphoreType.DMA((NBUF,)),
        ],
    )(nt.reshape(1), k.reshape(BH, 1, D), k_cache.reshape(BH, S, D))

    return (out_k.reshape(B, H, S, D), out_v.reshape(B, H, S, D))


# speedup vs baseline: 1.2576x; 1.1674x over previous
import jax
import jax.numpy as jnp
from jax.experimental import pallas as pl
from jax.experimental.pallas import tpu as pltpu

CBH = 2
NBUF = 12
DEPTH = 6


def _body(nt_ref, k_ref, kc_ref, v_ref, vc_ref, ok_ref, ov_ref,
          bufs, krows, vrows, sem_rows, sems_in, sems_out):
    BH, S, D = kc_ref.shape
    n_chunks = 2 * (BH // CBH)
    nt = nt_ref[0]

    ld_k = pltpu.make_async_copy(k_ref, krows, sem_rows)
    ld_v = pltpu.make_async_copy(v_ref, vrows, sem_rows)
    ld_k.start()
    ld_v.start()

    def chunk_refs(i):
        cache, bh = i % 2, (i // 2) * CBH
        src = kc_ref if cache == 0 else vc_ref
        dst = ok_ref if cache == 0 else ov_ref
        rows = krows if cache == 0 else vrows
        return src.at[pl.ds(bh, CBH)], dst.at[pl.ds(bh, CBH)], rows, bh

    def start_in(i):
        src, _, _, _ = chunk_refs(i)
        cp = pltpu.make_async_copy(src, bufs.at[i % NBUF], sems_in.at[i % NBUF])
        cp.start()
        return cp

    def drain(i, in_copies):
        _, dst, rows, bh = chunk_refs(i)
        slot = i % NBUF
        if i == 0:
            ld_k.wait()
            ld_v.wait()
        in_copies[i].wait()
        for c in range(CBH):
            bufs[slot, c, pl.ds(nt, 1), :] = rows[bh + c]
        cp = pltpu.make_async_copy(bufs.at[slot], dst, sems_out.at[slot])
        cp.start()
        return cp

    in_copies, out_copies = {}, {}
    for i in range(n_chunks):
        if i >= NBUF:
            out_copies[i - NBUF].wait()
        in_copies[i] = start_in(i)
        if i >= DEPTH:
            out_copies[i - DEPTH] = drain(i - DEPTH, in_copies)
    for j in range(n_chunks - DEPTH, n_chunks):
        out_copies[j] = drain(j, in_copies)
    for j in range(max(0, n_chunks - NBUF), n_chunks):
        out_copies[j].wait()


def kernel(k, k_cache, v, v_cache, n_tokens):
    B, H, S, D = k_cache.shape
    BH = B * H
    nt = jnp.asarray(n_tokens, jnp.int32).reshape(1)
    k2 = k.reshape(BH, 1, D)
    v2 = v.reshape(BH, 1, D)
    kc = k_cache.reshape(BH, S, D)
    vc = v_cache.reshape(BH, S, D)

    any_spec = pl.BlockSpec(memory_space=pl.ANY)
    out_k, out_v = pl.pallas_call(
        _body,
        in_specs=[
            pl.BlockSpec(memory_space=pltpu.SMEM),
            any_spec, any_spec, any_spec, any_spec,
        ],
        out_specs=[any_spec, any_spec],
        out_shape=[
            jax.ShapeDtypeStruct((BH, S, D), k_cache.dtype),
            jax.ShapeDtypeStruct((BH, S, D), v_cache.dtype),
        ],
        scratch_shapes=(
            [pltpu.VMEM((NBUF, CBH, S, D), k_cache.dtype),
             pltpu.VMEM((BH, 1, D), k.dtype),
             pltpu.VMEM((BH, 1, D), v.dtype),
             pltpu.SemaphoreType.DMA,
             pltpu.SemaphoreType.DMA((NBUF,)),
             pltpu.SemaphoreType.DMA((NBUF,))]
        ),
    )(nt, k2, kc, v2, vc)
    return (out_k.reshape(B, H, S, D), out_v.reshape(B, H, S, D))
